# baseline (speedup 1.0000x reference)
import functools

import jax
import jax.numpy as jnp
from jax import lax
from jax.experimental import pallas as pl
from jax.experimental.pallas import tpu as pltpu
from jax.experimental.pallas import tpu_sc as plsc

NB, BSZ, NG, BPG = 8, 4, 4, 2
L = 16
G = 4
NC, NS = 2, 16
NPS = 628
CH_MAX = NPS // G
CH_PAD = 160
BN = 80


def _sc_gather_body(N, K, D,
                    msg_f, idx_f, gout,
                    idx_all, buf, sem_g, sem_o, sem_i):
    bs = lax.axis_index("c")
    sid = lax.axis_index("s")
    n_start = sid * NPS
    nodes_here = jnp.minimum(NPS, N - n_start)
    ch_count = nodes_here // G

    row0 = (bs * NS + sid) * CH_PAD
    pltpu.async_copy(idx_f.at[pl.ds(row0, CH_PAD)], idx_all, sem_i)
    pltpu.make_async_copy(idx_f.at[pl.ds(0, CH_PAD)], idx_all, sem_i).wait()

    def obase(j):
        return (bs * N + n_start + j * G) * K

    def issue(j, p):
        @pl.when(j < ch_count)
        def _():
            pltpu.async_copy(msg_f.at[idx_all.at[j]], buf.at[p], sem_g.at[p])

    def wait_in(j, p):
        pltpu.make_async_copy(msg_f.at[idx_all.at[j]], buf.at[p],
                              sem_g.at[p]).wait()

    def wait_out(j, p):
        pltpu.make_async_copy(buf.at[p], gout.at[pl.ds(obase(j), G * K)],
                              sem_o.at[p]).wait()

    issue(0, 0)

    def chunk_iter(j, _):
        p = j % 2
        pn = (j + 1) % 2

        @pl.when((j >= 1) & (j - 1 < ch_count))
        def _():
            wait_out(j - 1, pn)

        issue(j + 1, pn)

        @pl.when(j < ch_count)
        def _():
            wait_in(j, p)
            pltpu.async_copy(buf.at[p], gout.at[pl.ds(obase(j), G * K)],
                             sem_o.at[p])

        return 0

    lax.fori_loop(0, CH_MAX, chunk_iter, 0)

    @pl.when(ch_count == CH_MAX)
    def _():
        wait_out(CH_MAX - 1, (CH_MAX - 1) % 2)


@functools.partial(jax.jit, static_argnums=(2, 3, 4))
def _sc_gather(msg_f, idx_f, N, K, D):
    mesh = plsc.VectorSubcoreMesh(core_axis_name="c", subcore_axis_name="s",
                                  num_cores=NC, num_subcores=NS)
    body = functools.partial(_sc_gather_body, N, K, D)
    BSN = msg_f.shape[0]
    return pl.kernel(
        body,
        out_type=jax.ShapeDtypeStruct((BSN * K, D), jnp.float32),
        mesh=mesh,
        compiler_params=pltpu.CompilerParams(needs_layout_passes=False),
        scratch_types=[
            pltpu.VMEM((CH_PAD, G * K), jnp.int32),
            pltpu.VMEM((2, G * K, D), jnp.float32),
            pltpu.SemaphoreType.DMA((2,)),
            pltpu.SemaphoreType.DMA((2,)),
            pltpu.SemaphoreType.DMA,
        ],
    )(msg_f, idx_f)


def _tc_math_body(NBLK, K, D, cc_ref, gout_ref, h_ref, key_ref, prim_ref,
                  dec_ref, bw_ref, gw_ref, hn_ref, mn_ref):
    i = pl.program_id(0)
    msgs = gout_ref[...].reshape(BN, K, D)
    key = key_ref[...]
    sim = jnp.sum(msgs * key[:, None, :], axis=-1)
    rt = jax.nn.sigmoid(sim)
    w = (msgs * rt[..., None]).reshape(BN, NB, BSZ, D)
    bw = bw_ref[...].astype(jnp.float32).reshape(BN, NB, BSZ, D)
    branch = jnp.tanh(jnp.sum(w * bw, axis=2))
    gw = gw_ref[...].astype(jnp.float32).reshape(BN, NG, BPG, D)
    group = jnp.tanh(jnp.sum(branch.reshape(BN, NG, BPG, D) * gw, axis=2))
    received = jnp.mean(group, axis=1)
    received = received + jnp.where((i % NBLK) == 0, cc_ref[0], 0.0)
    dec = dec_ref[...]
    hn = dec * h_ref[...] + (1.0 - dec) * received
    hn_ref[...] = hn
    mn_ref[...] = jnp.tanh(hn * prim_ref[...])


@functools.partial(jax.jit, static_argnums=(8, 9, 10))
def _tc_math(gout, h_f, key_f, prim_f, dec_f, bw_f, gw_f, cc80, N, K, D):
    NBLK = N // BN
    BSN = h_f.shape[0]
    grid = (BSN // BN,)
    body = functools.partial(_tc_math_body, NBLK, K, D)
    row = lambda i: (i, 0)
    wrow = lambda i: (i % NBLK, 0)
    return pl.pallas_call(
        body,
        grid=grid,
        in_specs=[
            pl.BlockSpec((1, BN, D), lambda i: (i // NBLK, 0, 0)),
            pl.BlockSpec((BN * K, D), row),
            pl.BlockSpec((BN, D), row),
            pl.BlockSpec((BN, D), row),
            pl.BlockSpec((BN, D), row),
            pl.BlockSpec((BN, D), row),
            pl.BlockSpec((BN * NB * BSZ, D), wrow),
            pl.BlockSpec((BN * NG * BPG, D), wrow),
        ],
        out_specs=[
            pl.BlockSpec((BN, D), row),
            pl.BlockSpec((BN, D), row),
        ],
        out_shape=[
            jax.ShapeDtypeStruct((BSN, D), jnp.float32),
            jax.ShapeDtypeStruct((BSN, D), jnp.float32),
        ],
    )(cc80, gout, h_f, key_f, prim_f, dec_f, bw_f, gw_f)


def kernel(cc_signals, h_prev, prev_messages, eff_prim, eff_key, eff_decay,
           conn_indices, branch_w, group_w):
    BS, T, C, D = cc_signals.shape
    N, K = conn_indices.shape
    n_pad = NS * NPS

    conn = conn_indices.astype(jnp.int32)
    conn = jnp.pad(conn, ((0, n_pad - N), (0, 0)))
    idx_f = (conn[None] + (jnp.arange(BS, dtype=jnp.int32) * N)[:, None, None])
    idx_f = idx_f.reshape(BS, NS, CH_MAX, G * K)
    idx_f = jnp.pad(idx_f, ((0, 0), (0, 0), (0, CH_PAD - CH_MAX), (0, 0)))
    idx_f = idx_f.reshape(BS * NS * CH_PAD, G * K)

    dec_f = jnp.broadcast_to(eff_decay[..., None], (BS, N, D)).reshape(BS * N, D)
    h_f = h_prev.reshape(BS * N, D)
    msg_f = prev_messages.reshape(BS * N, D)
    key_f = eff_key.reshape(BS * N, D)
    prim_f = eff_prim.reshape(BS * N, D)
    bw_f = branch_w.reshape(N * NB * BSZ, D).astype(jnp.bfloat16)
    gw_f = group_w.reshape(N * NG * BPG, D).astype(jnp.bfloat16)

    outs = []
    h, m = h_f, msg_f
    for t in range(T):
        cc80 = jnp.zeros((BS, BN, D), jnp.float32).at[:, :C].set(cc_signals[:, t])
        gout = _sc_gather(m, idx_f, N, K, D)
        h, m = _tc_math(gout, h, key_f, prim_f, dec_f, bw_f, gw_f, cc80,
                        N, K, D)
        outs.append(m.reshape(BS, N, D)[:, :C])

    output = jnp.stack(outs, axis=1)
    return output, h.reshape(BS, N, D)
